# baseline (device time: 12465 ns/iter reference)
import jax
import jax.numpy as jnp
from jax import lax
from jax.experimental import pallas as pl
from jax.experimental.pallas import tpu as pltpu

N_DEV = 8
BLK = 128


def kernel(x, w_mat):
    m_full, k_shard = x.shape
    k_full, n = w_mat.shape

    def body(x_ref, w_ref, out_ref, xb_ref, recv_ref, send_sems, recv_sems):
        my_i = lax.axis_index("i")

        barrier_sem = pltpu.get_barrier_semaphore()
        for s in range(1, N_DEV):
            pl.semaphore_signal(
                barrier_sem, inc=1,
                device_id=((my_i + s) % N_DEV,),
                device_id_type=pl.DeviceIdType.MESH,
            )
        pl.semaphore_wait(barrier_sem, N_DEV - 1)

        xb_ref[...] = x_ref[...].astype(jnp.bfloat16)

        sends = []
        for s in range(1, N_DEV):
            tgt = (my_i + s) % N_DEV
            rdma = pltpu.make_async_remote_copy(
                src_ref=xb_ref.at[pl.ds(tgt * BLK, BLK), :],
                dst_ref=recv_ref.at[s],
                send_sem=send_sems.at[s],
                recv_sem=recv_sems.at[s],
                device_id=(tgt,),
                device_id_type=pl.DeviceIdType.MESH,
            )
            rdma.start()
            sends.append(rdma)

        def wblk(k_idx):
            return w_ref[pl.ds(k_idx * BLK, BLK), :].astype(jnp.bfloat16)

        acc = lax.dot_general(
            xb_ref[pl.ds(my_i * BLK, BLK), :], wblk(my_i),
            (((1,), (0,)), ((), ())),
            preferred_element_type=jnp.float32,
        )

        for s in range(1, N_DEV):
            sends[s - 1].wait_recv()
            k_idx = (my_i - s) % N_DEV
            acc += lax.dot_general(
                recv_ref[s], wblk(k_idx),
                (((1,), (0,)), ((), ())),
                preferred_element_type=jnp.float32,
            )

        out_ref[...] = jnp.maximum(acc, 0.0)

        for rdma in sends:
            rdma.wait_send()

    return pl.pallas_call(
        body,
        out_shape=jax.ShapeDtypeStruct((m_full // N_DEV, n), jnp.float32),
        in_specs=[
            pl.BlockSpec(memory_space=pltpu.VMEM),
            pl.BlockSpec(memory_space=pltpu.VMEM),
        ],
        out_specs=pl.BlockSpec(memory_space=pltpu.VMEM),
        scratch_shapes=[
            pltpu.VMEM((m_full, k_shard), jnp.bfloat16),
            pltpu.VMEM((N_DEV, BLK, BLK), jnp.bfloat16),
            pltpu.SemaphoreType.DMA((N_DEV,)),
            pltpu.SemaphoreType.DMA((N_DEV,)),
        ],
        compiler_params=pltpu.CompilerParams(collective_id=0),
    )(x, w_mat)


# device time: 12405 ns/iter; 1.0048x vs baseline; 1.0048x over previous
import jax
import jax.numpy as jnp
from jax import lax
from jax.experimental import pallas as pl
from jax.experimental.pallas import tpu as pltpu

N_DEV = 8
BLK = 128


def kernel(x, w_mat):
    m_full, k_shard = x.shape
    k_full, n = w_mat.shape
    m_out = m_full // N_DEV

    def body(x_ref, w_ref, out_ref, xb_ref, xrow_ref, wb_ref,
             send_sems, recv_sems):
        my_i = lax.axis_index("i")

        barrier_sem = pltpu.get_barrier_semaphore()
        for s in range(1, N_DEV):
            pl.semaphore_signal(
                barrier_sem, inc=1,
                device_id=((my_i + s) % N_DEV,),
                device_id_type=pl.DeviceIdType.MESH,
            )
        pl.semaphore_wait(barrier_sem, N_DEV - 1)

        xb_ref[...] = x_ref[...].astype(jnp.bfloat16)

        sends = []
        for s in range(1, N_DEV):
            tgt = (my_i + s) % N_DEV
            rdma = pltpu.make_async_remote_copy(
                src_ref=xb_ref.at[pl.ds(tgt * BLK, BLK), :],
                dst_ref=xrow_ref.at[:, pl.ds(my_i * BLK, BLK)],
                send_sem=send_sems.at[s],
                recv_sem=recv_sems.at[s],
                device_id=(tgt,),
                device_id_type=pl.DeviceIdType.MESH,
            )
            rdma.start()
            sends.append(rdma)

        xrow_ref[:, pl.ds(my_i * BLK, BLK)] = xb_ref[pl.ds(my_i * BLK, BLK), :]

        wb_ref[...] = w_ref[...].astype(jnp.bfloat16)

        for rdma in sends:
            rdma.wait_recv()

        acc = lax.dot_general(
            xrow_ref[...], wb_ref[...],
            (((1,), (0,)), ((), ())),
            preferred_element_type=jnp.float32,
        )
        out_ref[...] = jnp.maximum(acc, 0.0)

        for rdma in sends:
            rdma.wait_send()

    return pl.pallas_call(
        body,
        out_shape=jax.ShapeDtypeStruct((m_out, n), jnp.float32),
        in_specs=[
            pl.BlockSpec(memory_space=pltpu.VMEM),
            pl.BlockSpec(memory_space=pltpu.VMEM),
        ],
        out_specs=pl.BlockSpec(memory_space=pltpu.VMEM),
        scratch_shapes=[
            pltpu.VMEM((m_full, k_shard), jnp.bfloat16),
            pltpu.VMEM((m_out, k_full), jnp.bfloat16),
            pltpu.VMEM((k_full, n), jnp.bfloat16),
            pltpu.SemaphoreType.DMA((N_DEV,)),
            pltpu.SemaphoreType.DMA((N_DEV,)),
        ],
        compiler_params=pltpu.CompilerParams(collective_id=0),
    )(x, w_mat)
